# Initial kernel scaffold; baseline (speedup 1.0000x reference)
import jax
import jax.numpy as jnp
from jax.experimental import pallas as pl


def kernel(embedding, seg_gt):
    raise NotImplementedError("write your pallas kernel here")


# speedup vs baseline: 1.3596x; 1.3596x over previous
import jax
import jax.numpy as jnp
from jax import lax
from jax.experimental import pallas as pl

EMBED_DIM = 16
DELTA_V = 0.5
DELTA_D = 3.0


def _safe_norm(sq):
    pos = sq > 0
    safe = jnp.where(pos, sq, 1.0)
    return jnp.sqrt(safe) * pos.astype(sq.dtype)


def _loss_kernel(e_ref, m_ref, var_ref, dist_ref, *, L, B):
    b = pl.program_id(0)

    e = e_ref[0]
    mf = m_ref[0].astype(jnp.float32)

    counts = jnp.sum(mf, axis=1, keepdims=True)
    sums = lax.dot_general(
        mf, e, (((1,), (1,)), ((), ())),
        preferred_element_type=jnp.float32)
    means = sums / counts

    enorm2 = jnp.sum(e * e, axis=0, keepdims=True)
    mnorm2 = jnp.sum(means * means, axis=1, keepdims=True)
    dot2 = lax.dot_general(
        means, e, (((1,), (0,)), ((), ())),
        preferred_element_type=jnp.float32)

    sq = jnp.maximum(enorm2 - 2.0 * dot2 + mnorm2, 0.0)
    norm = _safe_norm(sq)
    var_t = jnp.maximum(norm - DELTA_V, 0.0) ** 2 * mf
    var_per_lane = jnp.sum(var_t, axis=1, keepdims=True) / counts
    var_partial = jnp.sum(var_per_lane) / (L * B)

    cdiff = means[:, None, :] - means[None, :, :]
    dsq = jnp.sum(cdiff * cdiff, axis=2)
    eye = (lax.broadcasted_iota(jnp.int32, (L, L), 0)
           == lax.broadcasted_iota(jnp.int32, (L, L), 1)).astype(jnp.float32)
    dist = _safe_norm(dsq) + eye * DELTA_D
    dist_terms = jnp.maximum(DELTA_D - dist, 0.0) ** 2
    dist_partial = jnp.sum(dist_terms) / (L * (L - 1)) / 2.0 / B

    @pl.when(b == 0)
    def _():
        var_ref[:, :] = jnp.zeros((1, 1), jnp.float32)
        dist_ref[:, :] = jnp.zeros((1, 1), jnp.float32)

    var_ref[:, :] += var_partial.reshape(1, 1)
    dist_ref[:, :] += dist_partial.reshape(1, 1)


def kernel(embedding, seg_gt):
    B, D, H, W = embedding.shape
    L = seg_gt.shape[1]
    N = H * W

    e = embedding.reshape(B, D, N)
    m = seg_gt.reshape(B, L, N)

    var_loss, dist_loss = pl.pallas_call(
        lambda e_ref, m_ref, v_ref, d_ref: _loss_kernel(
            e_ref, m_ref, v_ref, d_ref, L=L, B=B),
        grid=(B,),
        in_specs=[
            pl.BlockSpec((1, D, N), lambda b: (b, 0, 0)),
            pl.BlockSpec((1, L, N), lambda b: (b, 0, 0)),
        ],
        out_specs=[
            pl.BlockSpec((1, 1), lambda b: (0, 0)),
            pl.BlockSpec((1, 1), lambda b: (0, 0)),
        ],
        out_shape=[
            jax.ShapeDtypeStruct((1, 1), jnp.float32),
            jax.ShapeDtypeStruct((1, 1), jnp.float32),
        ],
    )(e, m)

    reg_loss = jnp.zeros((), dtype=embedding.dtype)
    return (var_loss[0, 0], dist_loss[0, 0], reg_loss)
